# baseline (speedup 1.0000x reference)
import functools

import jax
import jax.numpy as jnp
from jax import lax
from jax.experimental import pallas as pl
from jax.experimental.pallas import tpu as pltpu
from jax.experimental.pallas import tpu_sc as plsc

_B, _N, _Fh, _Fv, _Fe = 1024, 1024, 64, 64, 16
_G = 4 * _Fh

_NC, _NS, _L = 2, 16, 16
_NW = _NC * _NS
_BPW = _B // _NW
_EPW = _BPW * _Fh


def _sc_gather_body(src_hbm, dst_hbm, h1_hbm, c1_hbm,
                    hs_out, cs_out, hd_out, cd_out,
                    sv, dv, sidx, didx, rsh, rsc, rdh, rdc,
                    ov0, ov1, ov2, ov3, sem):
    wid = lax.axis_index("s") * _NC + lax.axis_index("c")
    base = wid * _BPW
    cp_s = pltpu.async_copy(src_hbm.at[pl.ds(base, _BPW)], sv, sem)
    cp_d = pltpu.async_copy(dst_hbm.at[pl.ds(base, _BPW)], dv, sem)
    cp_s.wait()
    cp_d.wait()

    lane = lax.iota(jnp.int32, _L)

    def build(c, _):
        f = c // (_BPW // _L)
        half = c % (_BPW // _L)
        foff = (f // 8) * 8192 + (f % 8) * 128
        e = base + half * _L + lane
        ebase = e * (_N * _Fh) + foff
        s16 = sv[pl.ds(half * _L, _L)]
        d16 = dv[pl.ds(half * _L, _L)]
        sl = pl.ds(c * _L, _L)
        sidx[sl] = ebase + ((s16 >> 7) << 10) + (s16 & 127)
        didx[sl] = ebase + ((d16 >> 7) << 10) + (d16 & 127)
        return 0

    lax.fori_loop(0, _Fh * (_BPW // _L), build, 0, unroll=16)

    cps = [
        pltpu.async_copy(h1_hbm.at[sidx], rsh, sem),
        pltpu.async_copy(c1_hbm.at[sidx], rsc, sem),
        pltpu.async_copy(h1_hbm.at[didx], rdh, sem),
        pltpu.async_copy(c1_hbm.at[didx], rdc, sem),
    ]
    for cp in cps:
        cp.wait()

    out_sl = pl.ds(base * _Fh, _EPW)
    out_cps = []
    for rows, ovb, out in ((rsh, ov0, hs_out), (rsc, ov1, cs_out),
                           (rdh, ov2, hd_out), (rdc, ov3, cd_out)):
        def reorder(c, _):
            e = c // (_Fh // _L)
            fv = (c % (_Fh // _L)) * _L + lane
            vals = plsc.load_gather(rows, [fv * _BPW + e])
            ovb[pl.ds(c * _L, _L)] = vals
            return 0

        lax.fori_loop(0, _BPW * (_Fh // _L), reorder, 0, unroll=16)
        out_cps.append(pltpu.async_copy(ovb, out.at[out_sl], sem))
    for cp in out_cps:
        cp.wait()


@functools.lru_cache(maxsize=1)
def _sc_gather():
    return functools.partial(
        pl.kernel,
        mesh=plsc.VectorSubcoreMesh(core_axis_name="c", subcore_axis_name="s"),
        out_type=[jax.ShapeDtypeStruct((_B * _Fh,), jnp.float32)] * 4,
        scratch_types=[
            pltpu.VMEM((_BPW,), jnp.int32),
            pltpu.VMEM((_BPW,), jnp.int32),
            pltpu.VMEM((_EPW,), jnp.int32),
            pltpu.VMEM((_EPW,), jnp.int32),
            pltpu.VMEM((_EPW,), jnp.float32),
            pltpu.VMEM((_EPW,), jnp.float32),
            pltpu.VMEM((_EPW,), jnp.float32),
            pltpu.VMEM((_EPW,), jnp.float32),
            pltpu.VMEM((_EPW,), jnp.float32),
            pltpu.VMEM((_EPW,), jnp.float32),
            pltpu.VMEM((_EPW,), jnp.float32),
            pltpu.VMEM((_EPW,), jnp.float32),
            pltpu.SemaphoreType.DMA,
        ],
        compiler_params=pltpu.CompilerParams(
            use_tc_tiling_on_sc=False, needs_layout_passes=False),
    )(_sc_gather_body)


def _lstm_body(hs_ref, cs_ref, hd_ref, cd_ref, sf_ref, df_ref, xe_ref,
               w1s_ref, w2s_ref, bs_ref, w1d_ref, w2d_ref, bd_ref,
               hsrc_ref, csrc_ref, hdst_ref, cdst_ref):
    hs = hs_ref[...]
    cs = cs_ref[...]
    hd = hd_ref[...]
    cd = cd_ref[...]
    tail = jnp.concatenate([sf_ref[...], df_ref[...], xe_ref[...]], axis=1)

    def cell(x_head, h, c, w1_ref, w2_ref, b_ref):
        x = jnp.concatenate([x_head, tail], axis=1)
        g = (jnp.dot(x, w1_ref[...], preferred_element_type=jnp.float32)
             + jnp.dot(h, w2_ref[...], preferred_element_type=jnp.float32)
             + b_ref[...])
        i = jax.nn.sigmoid(g[:, 0:_Fh])
        f = jax.nn.sigmoid(g[:, _Fh:2 * _Fh])
        gg = jnp.tanh(g[:, 2 * _Fh:3 * _Fh])
        o = jax.nn.sigmoid(g[:, 3 * _Fh:4 * _Fh])
        c2 = f * c + i * gg
        h2 = o * jnp.tanh(c2)
        return h2, c2

    h2s, c2s = cell(hd, hs, cs, w1s_ref, w2s_ref, bs_ref)
    h2d, c2d = cell(hs, hd, cd, w1d_ref, w2d_ref, bd_ref)
    hsrc_ref[...] = h2s
    csrc_ref[...] = c2s
    hdst_ref[...] = h2d
    cdst_ref[...] = c2d


_lstm = pl.pallas_call(
    _lstm_body,
    out_shape=[jax.ShapeDtypeStruct((_B, _Fh), jnp.float32)] * 4,
)


_BB = 16


def _copy_body(s_ref, d_ref, hs_ref, hd_ref, cs_ref, cd_ref, h_ref, c_ref,
               hn_ref, cn_ref):
    n_iota = lax.broadcasted_iota(jnp.int32, (_BB, 1, _N), 2)
    s = s_ref[...][0][:, :, None]
    d = d_ref[...][0][:, :, None]
    ms = n_iota == s
    md = n_iota == d

    hs = hs_ref[...][0][:, :, None]
    hd = hd_ref[...][0][:, :, None]
    out = jnp.where(ms, hs, h_ref[...])
    out = jnp.where(md, hd, out)
    hn_ref[...] = out

    cs = cs_ref[...][0][:, :, None]
    cd = cd_ref[...][0][:, :, None]
    out = jnp.where(ms, cs, c_ref[...])
    out = jnp.where(md, cd, out)
    cn_ref[...] = out


_copy_scatter = pl.pallas_call(
    _copy_body,
    grid=(_B // _BB,),
    in_specs=[
        pl.BlockSpec((1, _BB, 1), lambda i: (i, 0, 0)),
        pl.BlockSpec((1, _BB, 1), lambda i: (i, 0, 0)),
        pl.BlockSpec((1, _BB, _Fh), lambda i: (i, 0, 0)),
        pl.BlockSpec((1, _BB, _Fh), lambda i: (i, 0, 0)),
        pl.BlockSpec((1, _BB, _Fh), lambda i: (i, 0, 0)),
        pl.BlockSpec((1, _BB, _Fh), lambda i: (i, 0, 0)),
        pl.BlockSpec((_BB, _Fh, _N), lambda i: (i, 0, 0)),
        pl.BlockSpec((_BB, _Fh, _N), lambda i: (i, 0, 0)),
    ],
    out_specs=[
        pl.BlockSpec((_BB, _Fh, _N), lambda i: (i, 0, 0)),
        pl.BlockSpec((_BB, _Fh, _N), lambda i: (i, 0, 0)),
    ],
    out_shape=[jax.ShapeDtypeStruct((_B, _Fh, _N), jnp.float32)] * 2,
)


def _phys1d(X):
    return (X.transpose(0, 2, 1)
            .reshape(_B, 8, 8, 8, 128)
            .transpose(0, 1, 3, 2, 4)
            .reshape(-1))


def kernel(E, Xe, Xv, H, C, Wih_in, Whh_in, bih_in, bhh_in,
           Wih_out, Whh_out, bih_out, bhh_out):
    src = E[:, 0]
    dst = E[:, 1]
    HT = H.transpose(0, 2, 1)
    CT = C.transpose(0, 2, 1)

    hs1, cs1, hd1, cd1 = _sc_gather()(src, dst, _phys1d(H), _phys1d(C))
    hs = hs1.reshape(_B, _Fh)
    cs = cs1.reshape(_B, _Fh)
    hd = hd1.reshape(_B, _Fh)
    cd = cd1.reshape(_B, _Fh)

    srcF = Xv[:, 0, :]
    dstF = Xv[:, 1, :]
    Hsrc, Csrc, Hdst, Cdst = _lstm(
        hs, cs, hd, cd, srcF, dstF, Xe,
        Wih_out.T, Whh_out.T, (bih_out + bhh_out).reshape(1, _G),
        Wih_in.T, Whh_in.T, (bih_in + bhh_in).reshape(1, _G),
    )

    nb = _B // _BB
    HnT, CnT = _copy_scatter(
        src.reshape(nb, _BB, 1), dst.reshape(nb, _BB, 1),
        Hsrc.reshape(nb, _BB, _Fh), Hdst.reshape(nb, _BB, _Fh),
        Csrc.reshape(nb, _BB, _Fh), Cdst.reshape(nb, _BB, _Fh),
        HT, CT,
    )
    return Hsrc, Hdst, HnT.transpose(0, 2, 1), CnT.transpose(0, 2, 1)


# speedup vs baseline: 1.0042x; 1.0042x over previous
import functools

import jax
import jax.numpy as jnp
from jax import lax
from jax.experimental import pallas as pl
from jax.experimental.pallas import tpu as pltpu
from jax.experimental.pallas import tpu_sc as plsc

_B, _N, _Fh, _Fv, _Fe = 1024, 1024, 64, 64, 16
_G = 4 * _Fh

_NC, _NS, _L = 2, 16, 16
_NW = _NC * _NS
_BPW = _B // _NW
_EPW = _BPW * _Fh


def _sc_gather_body(src_hbm, dst_hbm, h1_hbm, c1_hbm,
                    hs_out, cs_out, hd_out, cd_out,
                    sv, dv, sidx, didx, rsh, rsc, rdh, rdc,
                    ov0, ov1, ov2, ov3, sem):
    wid = lax.axis_index("s") * _NC + lax.axis_index("c")
    base = wid * _BPW
    cp_s = pltpu.async_copy(src_hbm.at[pl.ds(base, _BPW)], sv, sem)
    cp_d = pltpu.async_copy(dst_hbm.at[pl.ds(base, _BPW)], dv, sem)
    cp_s.wait()
    cp_d.wait()

    lane = lax.iota(jnp.int32, _L)

    def build(c, _):
        f = c // (_BPW // _L)
        half = c % (_BPW // _L)
        foff = (f // 8) * 8192 + (f % 8) * 128
        e = base + half * _L + lane
        ebase = e * (_N * _Fh) + foff
        s16 = sv[pl.ds(half * _L, _L)]
        d16 = dv[pl.ds(half * _L, _L)]
        sl = pl.ds(c * _L, _L)
        sidx[sl] = ebase + ((s16 >> 7) << 10) + (s16 & 127)
        didx[sl] = ebase + ((d16 >> 7) << 10) + (d16 & 127)
        return 0

    lax.fori_loop(0, _Fh * (_BPW // _L), build, 0, unroll=16)

    cps = [
        pltpu.async_copy(h1_hbm.at[sidx], rsh, sem),
        pltpu.async_copy(c1_hbm.at[sidx], rsc, sem),
        pltpu.async_copy(h1_hbm.at[didx], rdh, sem),
        pltpu.async_copy(c1_hbm.at[didx], rdc, sem),
    ]
    for cp in cps:
        cp.wait()

    out_sl = pl.ds(base * _Fh, _EPW)
    out_cps = []
    for rows, ovb, out in ((rsh, ov0, hs_out), (rsc, ov1, cs_out),
                           (rdh, ov2, hd_out), (rdc, ov3, cd_out)):
        def reorder(c, _):
            e = c // (_Fh // _L)
            fv = (c % (_Fh // _L)) * _L + lane
            vals = plsc.load_gather(rows, [fv * _BPW + e])
            ovb[pl.ds(c * _L, _L)] = vals
            return 0

        lax.fori_loop(0, _BPW * (_Fh // _L), reorder, 0, unroll=16)
        out_cps.append(pltpu.async_copy(ovb, out.at[out_sl], sem))
    for cp in out_cps:
        cp.wait()


@functools.lru_cache(maxsize=1)
def _sc_gather():
    return functools.partial(
        pl.kernel,
        mesh=plsc.VectorSubcoreMesh(core_axis_name="c", subcore_axis_name="s"),
        out_type=[jax.ShapeDtypeStruct((_B * _Fh,), jnp.float32)] * 4,
        scratch_types=[
            pltpu.VMEM((_BPW,), jnp.int32),
            pltpu.VMEM((_BPW,), jnp.int32),
            pltpu.VMEM((_EPW,), jnp.int32),
            pltpu.VMEM((_EPW,), jnp.int32),
            pltpu.VMEM((_EPW,), jnp.float32),
            pltpu.VMEM((_EPW,), jnp.float32),
            pltpu.VMEM((_EPW,), jnp.float32),
            pltpu.VMEM((_EPW,), jnp.float32),
            pltpu.VMEM((_EPW,), jnp.float32),
            pltpu.VMEM((_EPW,), jnp.float32),
            pltpu.VMEM((_EPW,), jnp.float32),
            pltpu.VMEM((_EPW,), jnp.float32),
            pltpu.SemaphoreType.DMA,
        ],
        compiler_params=pltpu.CompilerParams(
            use_tc_tiling_on_sc=False, needs_layout_passes=False),
    )(_sc_gather_body)


def _lstm_body(hs_ref, cs_ref, hd_ref, cd_ref, sf_ref, df_ref, xe_ref,
               w1s_ref, w2s_ref, bs_ref, w1d_ref, w2d_ref, bd_ref,
               hsrc_ref, csrc_ref, hdst_ref, cdst_ref):
    hs = hs_ref[...]
    cs = cs_ref[...]
    hd = hd_ref[...]
    cd = cd_ref[...]
    tail = jnp.concatenate([sf_ref[...], df_ref[...], xe_ref[...]], axis=1)

    def cell(x_head, h, c, w1_ref, w2_ref, b_ref):
        x = jnp.concatenate([x_head, tail], axis=1)
        g = (jnp.dot(x, w1_ref[...], preferred_element_type=jnp.float32)
             + jnp.dot(h, w2_ref[...], preferred_element_type=jnp.float32)
             + b_ref[...])
        i = jax.nn.sigmoid(g[:, 0:_Fh])
        f = jax.nn.sigmoid(g[:, _Fh:2 * _Fh])
        gg = jnp.tanh(g[:, 2 * _Fh:3 * _Fh])
        o = jax.nn.sigmoid(g[:, 3 * _Fh:4 * _Fh])
        c2 = f * c + i * gg
        h2 = o * jnp.tanh(c2)
        return h2, c2

    h2s, c2s = cell(hd, hs, cs, w1s_ref, w2s_ref, bs_ref)
    h2d, c2d = cell(hs, hd, cd, w1d_ref, w2d_ref, bd_ref)
    hsrc_ref[...] = h2s
    csrc_ref[...] = c2s
    hdst_ref[...] = h2d
    cdst_ref[...] = c2d


_lstm = pl.pallas_call(
    _lstm_body,
    out_shape=[jax.ShapeDtypeStruct((_B, _Fh), jnp.float32)] * 4,
)


_BB = 16


def _copy_body(s_ref, d_ref, hs_ref, cs_ref, hd_ref, cd_ref,
               sf_ref, df_ref, xe_ref,
               w1s_ref, w2s_ref, bs_ref, w1d_ref, w2d_ref, bd_ref,
               h_ref, c_ref,
               hn_ref, cn_ref, hsrc_ref, hdst_ref,
               vh_s, vc_s, vh_d, vc_d):
    i = pl.program_id(0)

    @pl.when(i == 0)
    def _lstm_step():
        hs = hs_ref[...]
        cs = cs_ref[...]
        hd = hd_ref[...]
        cd = cd_ref[...]
        tail = jnp.concatenate([sf_ref[...], df_ref[...], xe_ref[...]], axis=1)

        def cell(x_head, h, c, w1_ref, w2_ref, b_ref):
            x = jnp.concatenate([x_head, tail], axis=1)
            g = (jnp.dot(x, w1_ref[...], preferred_element_type=jnp.float32)
                 + jnp.dot(h, w2_ref[...], preferred_element_type=jnp.float32)
                 + b_ref[...])
            ii = jax.nn.sigmoid(g[:, 0:_Fh])
            f = jax.nn.sigmoid(g[:, _Fh:2 * _Fh])
            gg = jnp.tanh(g[:, 2 * _Fh:3 * _Fh])
            o = jax.nn.sigmoid(g[:, 3 * _Fh:4 * _Fh])
            c2 = f * c + ii * gg
            h2 = o * jnp.tanh(c2)
            return h2, c2

        h2s, c2s = cell(hd, hs, cs, w1s_ref, w2s_ref, bs_ref)
        h2d, c2d = cell(hs, hd, cd, w1d_ref, w2d_ref, bd_ref)
        vh_s[...] = h2s
        vc_s[...] = c2s
        vh_d[...] = h2d
        vc_d[...] = c2d

    strip = pl.ds(i * _BB, _BB)
    hsv = vh_s[strip, :]
    hdv = vh_d[strip, :]
    csv = vc_s[strip, :]
    cdv = vc_d[strip, :]
    hsrc_ref[...] = hsv
    hdst_ref[...] = hdv

    n_iota = lax.broadcasted_iota(jnp.int32, (_BB, 1, _N), 2)
    s = s_ref[...][0][:, :, None]
    d = d_ref[...][0][:, :, None]
    ms = n_iota == s
    md = n_iota == d

    out = jnp.where(ms, hsv[:, :, None], h_ref[...])
    out = jnp.where(md, hdv[:, :, None], out)
    hn_ref[...] = out

    out = jnp.where(ms, csv[:, :, None], c_ref[...])
    out = jnp.where(md, cdv[:, :, None], out)
    cn_ref[...] = out


_copy_scatter = pl.pallas_call(
    _copy_body,
    grid=(_B // _BB,),
    in_specs=[
        pl.BlockSpec((1, _BB, 1), lambda i: (i, 0, 0)),
        pl.BlockSpec((1, _BB, 1), lambda i: (i, 0, 0)),
        pl.BlockSpec((_B, _Fh), lambda i: (0, 0)),
        pl.BlockSpec((_B, _Fh), lambda i: (0, 0)),
        pl.BlockSpec((_B, _Fh), lambda i: (0, 0)),
        pl.BlockSpec((_B, _Fh), lambda i: (0, 0)),
        pl.BlockSpec((_B, _Fv), lambda i: (0, 0)),
        pl.BlockSpec((_B, _Fv), lambda i: (0, 0)),
        pl.BlockSpec((_B, _Fe), lambda i: (0, 0)),
        pl.BlockSpec((Fi := _Fh + 2 * _Fv + _Fe, _G), lambda i: (0, 0)),
        pl.BlockSpec((_Fh, _G), lambda i: (0, 0)),
        pl.BlockSpec((1, _G), lambda i: (0, 0)),
        pl.BlockSpec((Fi, _G), lambda i: (0, 0)),
        pl.BlockSpec((_Fh, _G), lambda i: (0, 0)),
        pl.BlockSpec((1, _G), lambda i: (0, 0)),
        pl.BlockSpec((_BB, _Fh, _N), lambda i: (i, 0, 0)),
        pl.BlockSpec((_BB, _Fh, _N), lambda i: (i, 0, 0)),
    ],
    out_specs=[
        pl.BlockSpec((_BB, _Fh, _N), lambda i: (i, 0, 0)),
        pl.BlockSpec((_BB, _Fh, _N), lambda i: (i, 0, 0)),
        pl.BlockSpec((_BB, _Fh), lambda i: (i, 0)),
        pl.BlockSpec((_BB, _Fh), lambda i: (i, 0)),
    ],
    out_shape=[jax.ShapeDtypeStruct((_B, _Fh, _N), jnp.float32)] * 2
    + [jax.ShapeDtypeStruct((_B, _Fh), jnp.float32)] * 2,
    scratch_shapes=[pltpu.VMEM((_B, _Fh), jnp.float32)] * 4,
)


def _phys1d(X):
    return (X.transpose(0, 2, 1)
            .reshape(_B, 8, 8, 8, 128)
            .transpose(0, 1, 3, 2, 4)
            .reshape(-1))


def kernel(E, Xe, Xv, H, C, Wih_in, Whh_in, bih_in, bhh_in,
           Wih_out, Whh_out, bih_out, bhh_out):
    src = E[:, 0]
    dst = E[:, 1]
    HT = H.transpose(0, 2, 1)
    CT = C.transpose(0, 2, 1)

    hs1, cs1, hd1, cd1 = _sc_gather()(src, dst, _phys1d(H), _phys1d(C))
    hs = hs1.reshape(_B, _Fh)
    cs = cs1.reshape(_B, _Fh)
    hd = hd1.reshape(_B, _Fh)
    cd = cd1.reshape(_B, _Fh)

    nb = _B // _BB
    HnT, CnT, Hsrc, Hdst = _copy_scatter(
        src.reshape(nb, _BB, 1), dst.reshape(nb, _BB, 1),
        hs, cs, hd, cd,
        Xv[:, 0, :], Xv[:, 1, :], Xe,
        Wih_out.T, Whh_out.T, (bih_out + bhh_out).reshape(1, _G),
        Wih_in.T, Whh_in.T, (bih_in + bhh_in).reshape(1, _G),
        HT, CT,
    )
    return Hsrc, Hdst, HnT.transpose(0, 2, 1), CnT.transpose(0, 2, 1)


# speedup vs baseline: 1.0066x; 1.0024x over previous
import functools

import jax
import jax.numpy as jnp
from jax import lax
from jax.experimental import pallas as pl
from jax.experimental.pallas import tpu as pltpu
from jax.experimental.pallas import tpu_sc as plsc

_B, _N, _Fh, _Fv, _Fe = 1024, 1024, 64, 64, 16
_G = 4 * _Fh

_NC, _NS, _L = 2, 16, 16
_NW = _NC * _NS
_BPW = _B // _NW
_EPW = _BPW * _Fh


def _sc_gather_body(src_hbm, dst_hbm, h1_hbm, c1_hbm,
                    hs_out, cs_out, hd_out, cd_out,
                    sv, dv, sidx, didx, rsh, rsc, rdh, rdc,
                    ov0, ov1, ov2, ov3, sem):
    wid = lax.axis_index("s") * _NC + lax.axis_index("c")
    base = wid * _BPW
    cp_s = pltpu.async_copy(src_hbm.at[pl.ds(base, _BPW)], sv, sem)
    cp_d = pltpu.async_copy(dst_hbm.at[pl.ds(base, _BPW)], dv, sem)
    cp_s.wait()
    cp_d.wait()

    lane = lax.iota(jnp.int32, _L)

    for half in range(_BPW // _L):
        e = base + half * _L + lane
        s16 = sv[pl.ds(half * _L, _L)]
        d16 = dv[pl.ds(half * _L, _L)]
        sbase = e * (_N * _Fh) + ((s16 >> 7) << 10) + (s16 & 127)
        dbase = e * (_N * _Fh) + ((d16 >> 7) << 10) + (d16 & 127)

        def build(f, _):
            foff = ((f >> 3) << 13) + ((f & 7) << 7)
            sl = pl.ds(f * _BPW + half * _L, _L)
            sidx[sl] = sbase + foff
            didx[sl] = dbase + foff
            return 0

        lax.fori_loop(0, _Fh, build, 0, unroll=16)

    cps = [
        pltpu.async_copy(h1_hbm.at[sidx], rsh, sem),
        pltpu.async_copy(c1_hbm.at[sidx], rsc, sem),
        pltpu.async_copy(h1_hbm.at[didx], rdh, sem),
        pltpu.async_copy(c1_hbm.at[didx], rdc, sem),
    ]
    for cp in cps:
        cp.wait()

    out_sl = pl.ds(base * _Fh, _EPW)
    out_cps = []
    for rows, ovb, out in ((rsh, ov0, hs_out), (rsc, ov1, cs_out),
                           (rdh, ov2, hd_out), (rdc, ov3, cd_out)):
        def reorder(c, _):
            e = c // (_Fh // _L)
            fv = (c % (_Fh // _L)) * _L + lane
            vals = plsc.load_gather(rows, [fv * _BPW + e])
            ovb[pl.ds(c * _L, _L)] = vals
            return 0

        lax.fori_loop(0, _BPW * (_Fh // _L), reorder, 0, unroll=16)
        out_cps.append(pltpu.async_copy(ovb, out.at[out_sl], sem))
    for cp in out_cps:
        cp.wait()


@functools.lru_cache(maxsize=1)
def _sc_gather():
    return functools.partial(
        pl.kernel,
        mesh=plsc.VectorSubcoreMesh(core_axis_name="c", subcore_axis_name="s"),
        out_type=[jax.ShapeDtypeStruct((_B * _Fh,), jnp.float32)] * 4,
        scratch_types=[
            pltpu.VMEM((_BPW,), jnp.int32),
            pltpu.VMEM((_BPW,), jnp.int32),
            pltpu.VMEM((_EPW,), jnp.int32),
            pltpu.VMEM((_EPW,), jnp.int32),
            pltpu.VMEM((_EPW,), jnp.float32),
            pltpu.VMEM((_EPW,), jnp.float32),
            pltpu.VMEM((_EPW,), jnp.float32),
            pltpu.VMEM((_EPW,), jnp.float32),
            pltpu.VMEM((_EPW,), jnp.float32),
            pltpu.VMEM((_EPW,), jnp.float32),
            pltpu.VMEM((_EPW,), jnp.float32),
            pltpu.VMEM((_EPW,), jnp.float32),
            pltpu.SemaphoreType.DMA,
        ],
        compiler_params=pltpu.CompilerParams(
            use_tc_tiling_on_sc=False, needs_layout_passes=False),
    )(_sc_gather_body)


def _lstm_body(hs_ref, cs_ref, hd_ref, cd_ref, sf_ref, df_ref, xe_ref,
               w1s_ref, w2s_ref, bs_ref, w1d_ref, w2d_ref, bd_ref,
               hsrc_ref, csrc_ref, hdst_ref, cdst_ref):
    hs = hs_ref[...]
    cs = cs_ref[...]
    hd = hd_ref[...]
    cd = cd_ref[...]
    tail = jnp.concatenate([sf_ref[...], df_ref[...], xe_ref[...]], axis=1)

    def cell(x_head, h, c, w1_ref, w2_ref, b_ref):
        x = jnp.concatenate([x_head, tail], axis=1)
        g = (jnp.dot(x, w1_ref[...], preferred_element_type=jnp.float32)
             + jnp.dot(h, w2_ref[...], preferred_element_type=jnp.float32)
             + b_ref[...])
        i = jax.nn.sigmoid(g[:, 0:_Fh])
        f = jax.nn.sigmoid(g[:, _Fh:2 * _Fh])
        gg = jnp.tanh(g[:, 2 * _Fh:3 * _Fh])
        o = jax.nn.sigmoid(g[:, 3 * _Fh:4 * _Fh])
        c2 = f * c + i * gg
        h2 = o * jnp.tanh(c2)
        return h2, c2

    h2s, c2s = cell(hd, hs, cs, w1s_ref, w2s_ref, bs_ref)
    h2d, c2d = cell(hs, hd, cd, w1d_ref, w2d_ref, bd_ref)
    hsrc_ref[...] = h2s
    csrc_ref[...] = c2s
    hdst_ref[...] = h2d
    cdst_ref[...] = c2d


_lstm = pl.pallas_call(
    _lstm_body,
    out_shape=[jax.ShapeDtypeStruct((_B, _Fh), jnp.float32)] * 4,
)


_BB = 16


def _copy_body(s_ref, d_ref, hs_ref, cs_ref, hd_ref, cd_ref,
               sf_ref, df_ref, xe_ref,
               w1s_ref, w2s_ref, bs_ref, w1d_ref, w2d_ref, bd_ref,
               h_ref, c_ref,
               hn_ref, cn_ref, hsrc_ref, hdst_ref,
               vh_s, vc_s, vh_d, vc_d):
    i = pl.program_id(0)

    @pl.when(i == 0)
    def _lstm_step():
        hs = hs_ref[...]
        cs = cs_ref[...]
        hd = hd_ref[...]
        cd = cd_ref[...]
        tail = jnp.concatenate([sf_ref[...], df_ref[...], xe_ref[...]], axis=1)

        def cell(x_head, h, c, w1_ref, w2_ref, b_ref):
            x = jnp.concatenate([x_head, tail], axis=1)
            g = (jnp.dot(x, w1_ref[...], preferred_element_type=jnp.float32)
                 + jnp.dot(h, w2_ref[...], preferred_element_type=jnp.float32)
                 + b_ref[...])
            ii = jax.nn.sigmoid(g[:, 0:_Fh])
            f = jax.nn.sigmoid(g[:, _Fh:2 * _Fh])
            gg = jnp.tanh(g[:, 2 * _Fh:3 * _Fh])
            o = jax.nn.sigmoid(g[:, 3 * _Fh:4 * _Fh])
            c2 = f * c + ii * gg
            h2 = o * jnp.tanh(c2)
            return h2, c2

        h2s, c2s = cell(hd, hs, cs, w1s_ref, w2s_ref, bs_ref)
        h2d, c2d = cell(hs, hd, cd, w1d_ref, w2d_ref, bd_ref)
        vh_s[...] = h2s
        vc_s[...] = c2s
        vh_d[...] = h2d
        vc_d[...] = c2d

    strip = pl.ds(i * _BB, _BB)
    hsv = vh_s[strip, :]
    hdv = vh_d[strip, :]
    csv = vc_s[strip, :]
    cdv = vc_d[strip, :]
    hsrc_ref[...] = hsv
    hdst_ref[...] = hdv

    n_iota = lax.broadcasted_iota(jnp.int32, (_BB, 1, _N), 2)
    s = s_ref[...][0][:, :, None]
    d = d_ref[...][0][:, :, None]
    ms = n_iota == s
    md = n_iota == d

    out = jnp.where(ms, hsv[:, :, None], h_ref[...])
    out = jnp.where(md, hdv[:, :, None], out)
    hn_ref[...] = out

    out = jnp.where(ms, csv[:, :, None], c_ref[...])
    out = jnp.where(md, cdv[:, :, None], out)
    cn_ref[...] = out


_copy_scatter = pl.pallas_call(
    _copy_body,
    grid=(_B // _BB,),
    in_specs=[
        pl.BlockSpec((1, _BB, 1), lambda i: (i, 0, 0)),
        pl.BlockSpec((1, _BB, 1), lambda i: (i, 0, 0)),
        pl.BlockSpec((_B, _Fh), lambda i: (0, 0)),
        pl.BlockSpec((_B, _Fh), lambda i: (0, 0)),
        pl.BlockSpec((_B, _Fh), lambda i: (0, 0)),
        pl.BlockSpec((_B, _Fh), lambda i: (0, 0)),
        pl.BlockSpec((_B, _Fv), lambda i: (0, 0)),
        pl.BlockSpec((_B, _Fv), lambda i: (0, 0)),
        pl.BlockSpec((_B, _Fe), lambda i: (0, 0)),
        pl.BlockSpec((Fi := _Fh + 2 * _Fv + _Fe, _G), lambda i: (0, 0)),
        pl.BlockSpec((_Fh, _G), lambda i: (0, 0)),
        pl.BlockSpec((1, _G), lambda i: (0, 0)),
        pl.BlockSpec((Fi, _G), lambda i: (0, 0)),
        pl.BlockSpec((_Fh, _G), lambda i: (0, 0)),
        pl.BlockSpec((1, _G), lambda i: (0, 0)),
        pl.BlockSpec((_BB, _Fh, _N), lambda i: (i, 0, 0)),
        pl.BlockSpec((_BB, _Fh, _N), lambda i: (i, 0, 0)),
    ],
    out_specs=[
        pl.BlockSpec((_BB, _Fh, _N), lambda i: (i, 0, 0)),
        pl.BlockSpec((_BB, _Fh, _N), lambda i: (i, 0, 0)),
        pl.BlockSpec((_BB, _Fh), lambda i: (i, 0)),
        pl.BlockSpec((_BB, _Fh), lambda i: (i, 0)),
    ],
    out_shape=[jax.ShapeDtypeStruct((_B, _Fh, _N), jnp.float32)] * 2
    + [jax.ShapeDtypeStruct((_B, _Fh), jnp.float32)] * 2,
    scratch_shapes=[pltpu.VMEM((_B, _Fh), jnp.float32)] * 4,
)


def _phys1d(X):
    return (X.transpose(0, 2, 1)
            .reshape(_B, 8, 8, 8, 128)
            .transpose(0, 1, 3, 2, 4)
            .reshape(-1))


def kernel(E, Xe, Xv, H, C, Wih_in, Whh_in, bih_in, bhh_in,
           Wih_out, Whh_out, bih_out, bhh_out):
    src = E[:, 0]
    dst = E[:, 1]
    HT = H.transpose(0, 2, 1)
    CT = C.transpose(0, 2, 1)

    hs1, cs1, hd1, cd1 = _sc_gather()(src, dst, _phys1d(H), _phys1d(C))
    hs = hs1.reshape(_B, _Fh)
    cs = cs1.reshape(_B, _Fh)
    hd = hd1.reshape(_B, _Fh)
    cd = cd1.reshape(_B, _Fh)

    nb = _B // _BB
    HnT, CnT, Hsrc, Hdst = _copy_scatter(
        src.reshape(nb, _BB, 1), dst.reshape(nb, _BB, 1),
        hs, cs, hd, cd,
        Xv[:, 0, :], Xv[:, 1, :], Xe,
        Wih_out.T, Whh_out.T, (bih_out + bhh_out).reshape(1, _G),
        Wih_in.T, Whh_in.T, (bih_in + bhh_in).reshape(1, _G),
        HT, CT,
    )
    return Hsrc, Hdst, HnT.transpose(0, 2, 1), CnT.transpose(0, 2, 1)


# speedup vs baseline: 1.0066x; 1.0000x over previous
import functools

import jax
import jax.numpy as jnp
from jax import lax
from jax.experimental import pallas as pl
from jax.experimental.pallas import tpu as pltpu
from jax.experimental.pallas import tpu_sc as plsc

_B, _N, _Fh, _Fv, _Fe = 1024, 1024, 64, 64, 16
_G = 4 * _Fh

_NC, _NS, _L = 2, 16, 16
_NW = _NC * _NS
_BPW = _B // _NW
_EPW = _BPW * _Fh


def _sc_gather_body(src_hbm, dst_hbm, h1_hbm, c1_hbm,
                    hs_out, cs_out, hd_out, cd_out,
                    sv, dv, sidx, didx, rsh, rsc, rdh, rdc,
                    ov0, ov1, ov2, ov3, sem):
    wid = lax.axis_index("s") * _NC + lax.axis_index("c")
    base = wid * _BPW
    cp_s = pltpu.async_copy(src_hbm.at[pl.ds(base, _BPW)], sv, sem)
    cp_d = pltpu.async_copy(dst_hbm.at[pl.ds(base, _BPW)], dv, sem)
    cp_s.wait()
    cp_d.wait()

    lane = lax.iota(jnp.int32, _L)

    for half in range(_BPW // _L):
        e = base + half * _L + lane
        s16 = sv[pl.ds(half * _L, _L)]
        d16 = dv[pl.ds(half * _L, _L)]
        sbase = e * (_N * _Fh) + ((s16 >> 7) << 10) + (s16 & 127)
        dbase = e * (_N * _Fh) + ((d16 >> 7) << 10) + (d16 & 127)

        def build(f, _):
            foff = ((f >> 3) << 13) + ((f & 7) << 7)
            sl = pl.ds(f * _BPW + half * _L, _L)
            sidx[sl] = sbase + foff
            didx[sl] = dbase + foff
            return 0

        lax.fori_loop(0, _Fh, build, 0, unroll=16)

    cps = [
        pltpu.async_copy(h1_hbm.at[sidx], rsh, sem),
        pltpu.async_copy(c1_hbm.at[sidx], rsc, sem),
        pltpu.async_copy(h1_hbm.at[didx], rdh, sem),
        pltpu.async_copy(c1_hbm.at[didx], rdc, sem),
    ]
    for cp in cps:
        cp.wait()

    out_sl = pl.ds(base * _Fh, _EPW)
    out_cps = []
    for rows, ovb, out in ((rsh, ov0, hs_out), (rsc, ov1, cs_out),
                           (rdh, ov2, hd_out), (rdc, ov3, cd_out)):
        def reorder(c, _):
            e = c // (_Fh // _L)
            fv = (c % (_Fh // _L)) * _L + lane
            vals = plsc.load_gather(rows, [fv * _BPW + e])
            ovb[pl.ds(c * _L, _L)] = vals
            return 0

        lax.fori_loop(0, _BPW * (_Fh // _L), reorder, 0, unroll=16)
        out_cps.append(pltpu.async_copy(ovb, out.at[out_sl], sem))
    for cp in out_cps:
        cp.wait()


@functools.lru_cache(maxsize=1)
def _sc_gather():
    return functools.partial(
        pl.kernel,
        mesh=plsc.VectorSubcoreMesh(core_axis_name="c", subcore_axis_name="s"),
        out_type=[jax.ShapeDtypeStruct((_B * _Fh,), jnp.float32)] * 4,
        scratch_types=[
            pltpu.VMEM((_BPW,), jnp.int32),
            pltpu.VMEM((_BPW,), jnp.int32),
            pltpu.VMEM((_EPW,), jnp.int32),
            pltpu.VMEM((_EPW,), jnp.int32),
            pltpu.VMEM((_EPW,), jnp.float32),
            pltpu.VMEM((_EPW,), jnp.float32),
            pltpu.VMEM((_EPW,), jnp.float32),
            pltpu.VMEM((_EPW,), jnp.float32),
            pltpu.VMEM((_EPW,), jnp.float32),
            pltpu.VMEM((_EPW,), jnp.float32),
            pltpu.VMEM((_EPW,), jnp.float32),
            pltpu.VMEM((_EPW,), jnp.float32),
            pltpu.SemaphoreType.DMA,
        ],
        compiler_params=pltpu.CompilerParams(
            use_tc_tiling_on_sc=False, needs_layout_passes=False),
    )(_sc_gather_body)


_BB = 16


def _copy_body(s_ref, d_ref, hs_ref, cs_ref, hd_ref, cd_ref,
               sf_ref, df_ref, xe_ref,
               w1s_ref, w2s_ref, bs_ref, w1d_ref, w2d_ref, bd_ref,
               h_ref, c_ref,
               hn_ref, cn_ref, hsrc_ref, hdst_ref,
               vh_s, vc_s, vh_d, vc_d):
    i = pl.program_id(0)

    @pl.when(i == 0)
    def _lstm_step():
        hs = hs_ref[...]
        cs = cs_ref[...]
        hd = hd_ref[...]
        cd = cd_ref[...]
        tail = jnp.concatenate([sf_ref[...], df_ref[...], xe_ref[...]], axis=1)

        def cell(x_head, h, c, w1_ref, w2_ref, b_ref):
            x = jnp.concatenate([x_head, tail], axis=1)
            g = (jnp.dot(x, w1_ref[...], preferred_element_type=jnp.float32)
                 + jnp.dot(h, w2_ref[...], preferred_element_type=jnp.float32)
                 + b_ref[...])
            ii = jax.nn.sigmoid(g[:, 0:_Fh])
            f = jax.nn.sigmoid(g[:, _Fh:2 * _Fh])
            gg = jnp.tanh(g[:, 2 * _Fh:3 * _Fh])
            o = jax.nn.sigmoid(g[:, 3 * _Fh:4 * _Fh])
            c2 = f * c + ii * gg
            h2 = o * jnp.tanh(c2)
            return h2, c2

        h2s, c2s = cell(hd, hs, cs, w1s_ref, w2s_ref, bs_ref)
        h2d, c2d = cell(hs, hd, cd, w1d_ref, w2d_ref, bd_ref)
        vh_s[...] = h2s
        vc_s[...] = c2s
        vh_d[...] = h2d
        vc_d[...] = c2d

    strip = pl.ds(i * _BB, _BB)
    hsv = vh_s[strip, :]
    hdv = vh_d[strip, :]
    csv = vc_s[strip, :]
    cdv = vc_d[strip, :]
    hsrc_ref[...] = hsv
    hdst_ref[...] = hdv

    n_iota = lax.broadcasted_iota(jnp.int32, (_BB, 1, _N), 2)
    s = s_ref[...][0][:, :, None]
    d = d_ref[...][0][:, :, None]
    ms = n_iota == s
    md = n_iota == d

    out = jnp.where(ms, hsv[:, :, None], h_ref[...])
    out = jnp.where(md, hdv[:, :, None], out)
    hn_ref[...] = out

    out = jnp.where(ms, csv[:, :, None], c_ref[...])
    out = jnp.where(md, cdv[:, :, None], out)
    cn_ref[...] = out


_copy_scatter = pl.pallas_call(
    _copy_body,
    grid=(_B // _BB,),
    in_specs=[
        pl.BlockSpec((1, _BB, 1), lambda i: (i, 0, 0)),
        pl.BlockSpec((1, _BB, 1), lambda i: (i, 0, 0)),
        pl.BlockSpec((_B, _Fh), lambda i: (0, 0)),
        pl.BlockSpec((_B, _Fh), lambda i: (0, 0)),
        pl.BlockSpec((_B, _Fh), lambda i: (0, 0)),
        pl.BlockSpec((_B, _Fh), lambda i: (0, 0)),
        pl.BlockSpec((_B, _Fv), lambda i: (0, 0)),
        pl.BlockSpec((_B, _Fv), lambda i: (0, 0)),
        pl.BlockSpec((_B, _Fe), lambda i: (0, 0)),
        pl.BlockSpec((Fi := _Fh + 2 * _Fv + _Fe, _G), lambda i: (0, 0)),
        pl.BlockSpec((_Fh, _G), lambda i: (0, 0)),
        pl.BlockSpec((1, _G), lambda i: (0, 0)),
        pl.BlockSpec((Fi, _G), lambda i: (0, 0)),
        pl.BlockSpec((_Fh, _G), lambda i: (0, 0)),
        pl.BlockSpec((1, _G), lambda i: (0, 0)),
        pl.BlockSpec((_BB, _Fh, _N), lambda i: (i, 0, 0)),
        pl.BlockSpec((_BB, _Fh, _N), lambda i: (i, 0, 0)),
    ],
    out_specs=[
        pl.BlockSpec((_BB, _Fh, _N), lambda i: (i, 0, 0)),
        pl.BlockSpec((_BB, _Fh, _N), lambda i: (i, 0, 0)),
        pl.BlockSpec((_BB, _Fh), lambda i: (i, 0)),
        pl.BlockSpec((_BB, _Fh), lambda i: (i, 0)),
    ],
    out_shape=[jax.ShapeDtypeStruct((_B, _Fh, _N), jnp.float32)] * 2
    + [jax.ShapeDtypeStruct((_B, _Fh), jnp.float32)] * 2,
    scratch_shapes=[pltpu.VMEM((_B, _Fh), jnp.float32)] * 4,
)


def _phys1d(X):
    return (X.transpose(0, 2, 1)
            .reshape(_B, 8, 8, 8, 128)
            .transpose(0, 1, 3, 2, 4)
            .reshape(-1))


def kernel(E, Xe, Xv, H, C, Wih_in, Whh_in, bih_in, bhh_in,
           Wih_out, Whh_out, bih_out, bhh_out):
    src = E[:, 0]
    dst = E[:, 1]
    HT = H.transpose(0, 2, 1)
    CT = C.transpose(0, 2, 1)

    hs1, cs1, hd1, cd1 = _sc_gather()(src, dst, _phys1d(H), _phys1d(C))
    hs = hs1.reshape(_B, _Fh)
    cs = cs1.reshape(_B, _Fh)
    hd = hd1.reshape(_B, _Fh)
    cd = cd1.reshape(_B, _Fh)

    nb = _B // _BB
    HnT, CnT, Hsrc, Hdst = _copy_scatter(
        src.reshape(nb, _BB, 1), dst.reshape(nb, _BB, 1),
        hs, cs, hd, cd,
        Xv[:, 0, :], Xv[:, 1, :], Xe,
        Wih_out.T, Whh_out.T, (bih_out + bhh_out).reshape(1, _G),
        Wih_in.T, Whh_in.T, (bih_in + bhh_in).reshape(1, _G),
        HT, CT,
    )
    return Hsrc, Hdst, HnT.transpose(0, 2, 1), CnT.transpose(0, 2, 1)
